# Initial kernel scaffold; baseline (speedup 1.0000x reference)
import jax
import jax.numpy as jnp
from jax.experimental import pallas as pl


def kernel(x, edge_index, Wl0, bl0, Wr0, br0, att0, bias0, g0, b0, Wl1, bl1, Wr1, br1, att1, bias1, g1, b1, Wc, bc):
    raise NotImplementedError("write your pallas kernel here")


# speedup vs baseline: 1.9749x; 1.9749x over previous
import jax
import jax.numpy as jnp
from jax import lax
from jax.experimental import pallas as pl
from jax.experimental.pallas import tpu as pltpu
from jax.experimental.pallas import tpu_sc as plsc

N = 10000
E = 320000
H = 8
C = 32
NEG = 0.2
BN_EPS = 1e-5

NC = 2
NS = 16
L = 16
HC2 = 128
EPT = E // NS
K = 16
NCHUNK = EPT // K
NPAD = 10240
AROWS = 2048
GROWS = 128
RSPAN = AROWS - GROWS
ROUNDS = tuple((i * RSPAN, RSPAN) for i in range(5)) + ((5 * RSPAN, NPAD - 5 * RSPAN),)
MID_BLK = 1024
OUT_BLK = 1000
EBLK = 4000


def _sum_body(xl_hbm, xr_hbm, src_hbm, dst_hbm,
              erows_out,
              src_v, dst_v, idxa_v, idxb_v, xlr, xrr, sem1, sem2):
    c = lax.axis_index("c")
    s = lax.axis_index("s")
    cN = c * NPAD
    ebase0 = s * EPT

    @pl.loop(0, NCHUNK)
    def chunk(ci):
        eb = ebase0 + ci * K
        pltpu.sync_copy(src_hbm.at[pl.ds(eb, K)], src_v)
        pltpu.sync_copy(dst_hbm.at[pl.ds(eb, K)], dst_v)
        idxa_v[pl.ds(0, L)] = src_v[pl.ds(0, L)] + cN
        idxb_v[pl.ds(0, L)] = dst_v[pl.ds(0, L)] + cN
        cpa = pltpu.async_copy(xl_hbm.at[idxa_v], xlr, sem1)
        cpb = pltpu.async_copy(xr_hbm.at[idxb_v], xrr, sem2)
        cpa.wait()
        cpb.wait()

        def add_body(i, _):
            r = i // 8
            col = (i - 8 * r) * L
            xlr[r, pl.ds(col, L)] = (xlr[r, pl.ds(col, L)]
                                     + xrr[r, pl.ds(col, L)])
            return 0
        lax.fori_loop(0, K * 8, add_body, 0)

        pltpu.sync_copy(xlr, erows_out.at[c, pl.ds(eb, K), :])


def _agg_body(xl_hbm, exg_hbm, src_hbm, dst_hbm,
              msg_out, den_out,
              macc, dacc,
              src_v, dst_v, idxa_v, idxb_v, xlr, exr, sem1):
    c = lax.axis_index("c")
    s = lax.axis_index("s")
    cN = c * NPAD
    ebase0 = s * EPT

    def zero_bufs():
        zvec = jnp.zeros((L,), jnp.float32)

        def xz(i, _):
            xlr[i // 8, pl.ds((i % 8) * L, L)] = zvec
            return 0
        lax.fori_loop(0, K * 8, xz, 0)

    def row_chunks(rows):
        out, off = [], 0
        while off < rows:
            nr = min(K, rows - off)
            out.append((off, nr))
            off += nr
        return out

    for base, span in ROUNDS:
        zero_bufs()
        r0 = s * (AROWS // NS)
        for off, nr in row_chunks(AROWS // NS):
            pltpu.sync_copy(xlr.at[pl.ds(0, nr), :],
                            macc.at[pl.ds(r0 + off, nr), :])
            pltpu.sync_copy(xlr.at[pl.ds(0, nr), :],
                            dacc.at[pl.ds(r0 + off, nr), :])
        plsc.subcore_barrier()

        @pl.loop(0, NCHUNK)
        def chunk(ci):
            eb = ebase0 + ci * K
            pltpu.sync_copy(src_hbm.at[pl.ds(eb, K)], src_v)
            pltpu.sync_copy(dst_hbm.at[pl.ds(eb, K)], dst_v)
            idxa_v[pl.ds(0, L)] = src_v[pl.ds(0, L)] + cN
            cpa = pltpu.async_copy(xl_hbm.at[idxa_v], xlr, sem1)
            pltpu.sync_copy(exg_hbm.at[c, pl.ds(eb, K), :], exr)

            grow = RSPAN + s * 8 + (ci % 8)
            d = dst_v[pl.ds(0, L)] - base
            oob = (d < 0) | (d >= span)
            idxb_v[0, pl.ds(0, L)] = jnp.where(
                oob, jnp.full((L,), grow, jnp.int32), d)

            cpa.wait()

            def mul_body(i, _):
                r = i // 8
                col = (i - 8 * r) * L
                xlr[r, pl.ds(col, L)] = (xlr[r, pl.ds(col, L)]
                                         * exr[r, pl.ds(col, L)])
                return 0
            lax.fori_loop(0, K * 8, mul_body, 0)

            pltpu.sync_copy(xlr, macc.at[idxb_v.at[0]], add=True)
            pltpu.sync_copy(exr, dacc.at[idxb_v.at[0]], add=True)

        plsc.subcore_barrier()
        rows = span // NS
        r0d = s * rows
        for off, nr in row_chunks(rows):
            pltpu.sync_copy(macc.at[pl.ds(r0d + off, nr), :],
                            xlr.at[pl.ds(0, nr), :])
            pltpu.sync_copy(xlr.at[pl.ds(0, nr), :],
                            msg_out.at[c, pl.ds(base + r0d + off, nr), :])
            pltpu.sync_copy(dacc.at[pl.ds(r0d + off, nr), :],
                            exr.at[pl.ds(0, nr), :])
            pltpu.sync_copy(exr.at[pl.ds(0, nr), :],
                            den_out.at[c, pl.ds(base + r0d + off, nr), :])
        plsc.subcore_barrier()


_MESH = plsc.VectorSubcoreMesh(core_axis_name="c", subcore_axis_name="s",
                               num_cores=NC, num_subcores=NS)

_SUM_CALL = pl.kernel(
    _sum_body,
    out_type=[jax.ShapeDtypeStruct((NC, E, HC2), jnp.float32)],
    mesh=_MESH,
    scratch_types=[
        pltpu.VMEM((K,), jnp.int32),
        pltpu.VMEM((K,), jnp.int32),
        pltpu.VMEM((K,), jnp.int32),
        pltpu.VMEM((K,), jnp.int32),
        pltpu.VMEM((K, HC2), jnp.float32),
        pltpu.VMEM((K, HC2), jnp.float32),
        pltpu.SemaphoreType.DMA,
        pltpu.SemaphoreType.DMA,
    ],
)

_AGG_CALL = pl.kernel(
    _agg_body,
    out_type=[
        jax.ShapeDtypeStruct((NC, NPAD, HC2), jnp.float32),
        jax.ShapeDtypeStruct((NC, NPAD, HC2), jnp.float32),
    ],
    mesh=_MESH,
    scratch_types=[
        pltpu.VMEM_SHARED((AROWS, HC2), jnp.float32),
        pltpu.VMEM_SHARED((AROWS, HC2), jnp.float32),
        pltpu.VMEM((K,), jnp.int32),
        pltpu.VMEM((K,), jnp.int32),
        pltpu.VMEM((K,), jnp.int32),
        pltpu.VMEM((8, K), jnp.int32),
        pltpu.VMEM((K, HC2), jnp.float32),
        pltpu.VMEM((K, HC2), jnp.float32),
        pltpu.SemaphoreType.DMA,
    ],
)


def _proj_body(x_ref, w_ref, b_ref, o_ref):
    o_ref[0, 0] = (jnp.dot(x_ref[...], w_ref[0, 0],
                           preferred_element_type=jnp.float32)
                   + b_ref[0, 0, 0][None, :])


def _alpha_body(er, amat, o):
    for g in range(2):
        e = er[g]
        e = jnp.where(e > 0, e, NEG * e)
        o[g] = jnp.dot(e, amat[g], preferred_element_type=jnp.float32)


def _expand_body(al, gmax, exg):
    nb = al.shape[1]
    for g in range(2):
        ex = jnp.exp(al[g] - gmax[0, g][None, :])
        exg[g] = jnp.concatenate(
            [jnp.broadcast_to(ex[:, h:h + 1], (nb, C)) for h in range(4)],
            axis=1)


def _den_expand(d):
    return jnp.concatenate(
        [jnp.broadcast_to(d[:, h:h + 1], (d.shape[0], C)) for h in range(4)],
        axis=1)


def _mid_body(raw, den, bias0, s0, t0, wl, bl, wr, br, xl1, xr1):
    hs = []
    for g in range(2):
        h0 = raw[g] / (den[g] + 1e-16) + bias0[g][None, :]
        h0 = h0 * s0[g][None, :] + t0[g][None, :]
        h0 = jnp.where(h0 > 0, h0, jnp.exp(h0) - 1.0)
        hs.append(h0)
    for gp in range(2):
        acc_l = jnp.broadcast_to(bl[gp][None, :], (hs[0].shape[0], HC2))
        acc_r = jnp.broadcast_to(br[gp][None, :], (hs[0].shape[0], HC2))
        for g in range(2):
            acc_l = acc_l + jnp.dot(hs[g], wl[g, :, gp],
                                    preferred_element_type=jnp.float32)
            acc_r = acc_r + jnp.dot(hs[g], wr[g, :, gp],
                                    preferred_element_type=jnp.float32)
        xl1[gp] = acc_l
        xr1[gp] = acc_r


def _out_body(raw, den, bias1, s1, t1, wc, bc, o):
    acc = jnp.zeros((OUT_BLK, C), jnp.float32)
    for g in range(2):
        o1 = raw[g] / (den[g] + 1e-16)
        acc = acc + (o1[:, 0:32] + o1[:, 32:64] + o1[:, 64:96] + o1[:, 96:128])
    acc = acc / 8.0 + bias1[...][None, :]
    acc = acc * s1[...][None, :] + t1[...][None, :]
    o[...] = (jnp.dot(acc, wc[...], preferred_element_type=jnp.float32)
              + bc[...][None, :])


def _full(shape):
    return pl.BlockSpec(shape, lambda *_: tuple(0 for _ in shape))


def _edge_layer(xl_pair, xr_pair, src_e, dst_e, att):
    f32 = jnp.float32
    erows, = _SUM_CALL(xl_pair, xr_pair, src_e, dst_e)

    amat = jnp.zeros((2, HC2, 4), f32)
    for g in range(2):
        for h in range(4):
            amat = amat.at[g, h * C:(h + 1) * C, h].set(att[4 * g + h])

    alpha = pl.pallas_call(
        _alpha_body,
        grid=(E // EBLK,),
        in_specs=[
            pl.BlockSpec((NC, EBLK, HC2), lambda i: (0, i, 0)),
            _full((NC, HC2, 4)),
        ],
        out_specs=pl.BlockSpec((NC, EBLK, 4), lambda i: (0, i, 0)),
        out_shape=jax.ShapeDtypeStruct((NC, E, 4), f32),
    )(erows, amat)

    gmax = jnp.max(alpha, axis=1).reshape(1, NC, 4)

    exg = pl.pallas_call(
        _expand_body,
        grid=(E // EBLK,),
        in_specs=[
            pl.BlockSpec((NC, EBLK, 4), lambda i: (0, i, 0)),
            _full((1, NC, 4)),
        ],
        out_specs=pl.BlockSpec((NC, EBLK, HC2), lambda i: (0, i, 0)),
        out_shape=jax.ShapeDtypeStruct((NC, E, HC2), f32),
    )(alpha, gmax)

    msg, den = _AGG_CALL(xl_pair, exg, src_e, dst_e)
    return msg, den


def kernel(x, edge_index, Wl0, bl0, Wr0, br0, att0, bias0, g0, b0,
           Wl1, bl1, Wr1, br1, att1, bias1, g1, b1, Wc, bc):
    f32 = jnp.float32
    bns = float(1.0 / (1.0 + BN_EPS) ** 0.5)

    W0 = jnp.stack([
        jnp.stack([Wl0[:, :HC2], Wl0[:, HC2:]]),
        jnp.stack([Wr0[:, :HC2], Wr0[:, HC2:]]),
    ])
    b0p = jnp.stack([
        jnp.stack([bl0[:HC2], bl0[HC2:]]),
        jnp.stack([br0[:HC2], br0[HC2:]]),
    ]).reshape(2, 2, 1, HC2)
    xpad = jnp.pad(x, ((0, NPAD - N), (0, 0)))
    proj0 = pl.pallas_call(
        _proj_body,
        grid=(2, 2),
        in_specs=[
            pl.BlockSpec((NPAD, 128), lambda i, j: (0, 0)),
            pl.BlockSpec((1, 1, 128, HC2), lambda i, j: (i, j, 0, 0)),
            pl.BlockSpec((1, 1, 1, HC2), lambda i, j: (i, j, 0, 0)),
        ],
        out_specs=pl.BlockSpec((1, 1, NPAD, HC2), lambda i, j: (i, j, 0, 0)),
        out_shape=jax.ShapeDtypeStruct((2, 2, NPAD, HC2), f32),
    )(xpad, W0, b0p)

    src_e = edge_index[0]
    dst_e = edge_index[1]

    msg0, den0 = _edge_layer(proj0[0].reshape(NC * NPAD, HC2),
                             proj0[1].reshape(NC * NPAD, HC2),
                             src_e, dst_e, att0)

    grid_mid = NPAD // MID_BLK
    mid_specs = [
        pl.BlockSpec((NC, MID_BLK, HC2), lambda i: (0, i, 0)),
        pl.BlockSpec((NC, MID_BLK, HC2), lambda i: (0, i, 0)),
        _full((2, HC2)), _full((2, HC2)), _full((2, HC2)),
        _full((2, HC2, 2, HC2)), _full((2, HC2)),
        _full((2, HC2, 2, HC2)), _full((2, HC2)),
    ]
    xl1, xr1 = pl.pallas_call(
        _mid_body,
        grid=(grid_mid,),
        in_specs=mid_specs,
        out_specs=[pl.BlockSpec((NC, MID_BLK, HC2), lambda i: (0, i, 0))] * 2,
        out_shape=[jax.ShapeDtypeStruct((NC, NPAD, HC2), f32)] * 2,
    )(msg0, den0,
      bias0.reshape(2, HC2), (g0 * bns).reshape(2, HC2), b0.reshape(2, HC2),
      Wl1.reshape(2, HC2, 2, HC2), bl1.reshape(2, HC2),
      Wr1.reshape(2, HC2, 2, HC2), br1.reshape(2, HC2))

    msg1, den1 = _edge_layer(xl1.reshape(NC * NPAD, HC2),
                             xr1.reshape(NC * NPAD, HC2),
                             src_e, dst_e, att1)

    out = pl.pallas_call(
        _out_body,
        grid=(N // OUT_BLK,),
        in_specs=[
            pl.BlockSpec((NC, OUT_BLK, HC2), lambda i: (0, i, 0)),
            pl.BlockSpec((NC, OUT_BLK, HC2), lambda i: (0, i, 0)),
            _full((C,)), _full((C,)), _full((C,)),
            _full((C, 2)), _full((2,)),
        ],
        out_specs=pl.BlockSpec((OUT_BLK, 2), lambda i: (i, 0)),
        out_shape=jax.ShapeDtypeStruct((N, 2), f32),
    )(msg1, den1, bias1, g1 * bns, b1, Wc, bc)

    return out
